# baseline (speedup 1.0000x reference)
import functools

import numpy as np

import jax
import jax.numpy as jnp
from jax import lax
from jax.experimental import pallas as pl
from jax.experimental.pallas import tpu as pltpu
from jax.experimental.pallas import tpu_sc as plsc

N = 10000
E = 320000
D = 128
H = 128
EA = 4
W = 144
NB_N = 10
BN = N // NB_N
SPL = 2
EH = E // SPL
BE = 3200
NB_E = EH // BE

PERM = np.concatenate(
    [np.concatenate([np.arange(16) * 2 + 32 * k,
                     np.arange(16) * 2 + 1 + 32 * k]) for k in range(4)])

NC = 2
NS = 16
NW = NC * NS
EPW = EH // NW
CH = 40
NCH = EPW // CH
CPS = EH // CH


def _silu(v):
    return v * jax.nn.sigmoid(v)


def _proj_body(h_ref, wa_ref, wb_ref, pd_ref, ps_ref):
    hb = h_ref[...]
    pd_ref[...] = jnp.dot(
        hb, wa_ref[...], preferred_element_type=jnp.float32
    ).astype(jnp.bfloat16)
    ps_ref[...] = jnp.dot(
        hb, wb_ref[...], preferred_element_type=jnp.float32
    ).astype(jnp.bfloat16)


def _project(h, we1a, we1b):
    return pl.pallas_call(
        _proj_body,
        grid=(NB_N,),
        in_specs=[
            pl.BlockSpec((BN, D), lambda i: (i, 0)),
            pl.BlockSpec((D, H), lambda i: (0, 0)),
            pl.BlockSpec((D, H), lambda i: (0, 0)),
        ],
        out_specs=[
            pl.BlockSpec((BN, H), lambda i: (i, 0)),
            pl.BlockSpec((BN, H), lambda i: (i, 0)),
        ],
        out_shape=[
            jax.ShapeDtypeStruct((N, H), jnp.bfloat16),
            jax.ShapeDtypeStruct((N, H), jnp.bfloat16),
        ],
    )(h, we1a, we1b)


def _gather_body(crow0, td_hbm, ts_hbm, xq_hbm, dst2_hbm, src2_hbm, wdp_hbm,
                 go_hbm, rq_hbm,
                 dia, sia, wdb,
                 dbuf0, sbuf0, xd0, xs0, obuf0, rbuf0,
                 dbuf1, sbuf1, xd1, xs1, obuf1, rbuf1,
                 gsem0, gsem1, wsem0, wsem1):
    wid = lax.axis_index("s") * NC + lax.axis_index("c")
    base = wid * EPW
    pltpu.sync_copy(dst2_hbm.at[pl.ds(crow0 + wid * NCH, NCH)], dia)
    pltpu.sync_copy(src2_hbm.at[pl.ds(crow0 + wid * NCH, NCH)], sia)
    pltpu.sync_copy(wdp_hbm, wdb)

    bufs = [(dbuf0, sbuf0, xd0, xs0, obuf0, rbuf0, gsem0, wsem0),
            (dbuf1, sbuf1, xd1, xs1, obuf1, rbuf1, gsem1, wsem1)]

    def fire(i, b):
        db, sb, xd, xs, _, _, gs, _ = bufs[b]
        di = dia.at[i]
        si = sia.at[i]
        pltpu.async_copy(td_hbm.at[di], db, gs)
        pltpu.async_copy(ts_hbm.at[si], sb, gs)
        pltpu.async_copy(xq_hbm.at[di], xd, gs)
        pltpu.async_copy(xq_hbm.at[si], xs, gs)

    def drain_g(b):
        db, sb, xd, xs, _, _, gs, _ = bufs[b]
        pltpu.make_async_copy(td_hbm.at[pl.ds(0, CH)], db, gs).wait()
        pltpu.make_async_copy(ts_hbm.at[pl.ds(0, CH)], sb, gs).wait()
        pltpu.make_async_copy(xq_hbm.at[pl.ds(0, CH)], xd, gs).wait()
        pltpu.make_async_copy(xq_hbm.at[pl.ds(0, CH)], xs, gs).wait()

    def drain_w(b):
        _, _, _, _, ob, rb, _, ws = bufs[b]
        pltpu.make_async_copy(ob, go_hbm.at[pl.ds(0, CH)], ws).wait()
        pltpu.make_async_copy(rb, rq_hbm.at[pl.ds(0, CH)], ws).wait()

    def compute(b):
        db, sb, xd, xs, ob, rb, _, _ = bufs[b]

        himask = jnp.int32(-65536)

        def row(r, c2):
            rel = xd[r, :] - xs[r, :]
            rb[r, :] = rel
            ds = jnp.maximum(jnp.sum(rel * rel), 1e-8)
            dsv = jnp.full((16,), ds, jnp.float32)
            for k in range(4):
                vd = plsc.bitcast(db[r, pl.ds(k * 32, 32)], jnp.int32)
                vs = plsc.bitcast(sb[r, pl.ds(k * 32, 32)], jnp.int32)
                ob[r, pl.ds(k * 32, 16)] = (
                    plsc.bitcast(vd << 16, jnp.float32)
                    + plsc.bitcast(vs << 16, jnp.float32)
                    + dsv * wdb[pl.ds(k * 32, 16)])
                ob[r, pl.ds(k * 32 + 16, 16)] = (
                    plsc.bitcast(vd & himask, jnp.float32)
                    + plsc.bitcast(vs & himask, jnp.float32)
                    + dsv * wdb[pl.ds(k * 32 + 16, 16)])
            return c2

        lax.fori_loop(0, CH, row, 0, unroll=2)

    def write(i, b):
        _, _, _, _, ob, rb, _, ws = bufs[b]
        e0 = base + i * CH
        pltpu.async_copy(ob, go_hbm.at[pl.ds(e0, CH)], ws)
        pltpu.async_copy(rb, rq_hbm.at[pl.ds(e0, CH)], ws)

    fire(0, 0)

    @pl.loop(0, (NCH - 1) // 2)
    def body(p):
        i0 = 2 * p
        fire(i0 + 1, 1)
        drain_g(0)

        @pl.when(i0 >= 2)
        def _():
            drain_w(0)

        compute(0)
        write(i0, 0)
        fire(i0 + 2, 0)
        drain_g(1)

        @pl.when(i0 >= 1)
        def _():
            drain_w(1)

        compute(1)
        write(i0 + 1, 1)

    drain_g(0)
    drain_w(0)
    compute(0)
    write(NCH - 1, 0)
    drain_w(0)
    drain_w(1)


def _sc_gather(td, ts, xq, dst, src, wdp, crow0):
    mesh = plsc.VectorSubcoreMesh(core_axis_name="c", subcore_axis_name="s")
    f = pl.kernel(
        functools.partial(_gather_body, crow0),
        mesh=mesh,
        compiler_params=pltpu.CompilerParams(use_tc_tiling_on_sc=False,
                                             needs_layout_passes=False),
        out_type=[
            jax.ShapeDtypeStruct((EH, H), jnp.float32),
            jax.ShapeDtypeStruct((EH, 16), jnp.float32),
        ],
        scratch_types=[
            pltpu.VMEM((NCH, CH), jnp.int32),
            pltpu.VMEM((NCH, CH), jnp.int32),
            pltpu.VMEM((H,), jnp.float32),
        ] + 2 * [
            pltpu.VMEM((CH, H), jnp.bfloat16),
            pltpu.VMEM((CH, H), jnp.bfloat16),
            pltpu.VMEM((CH, 16), jnp.float32),
            pltpu.VMEM((CH, 16), jnp.float32),
            pltpu.VMEM((CH, H), jnp.float32),
            pltpu.VMEM((CH, 16), jnp.float32),
        ] + 4 * [pltpu.SemaphoreType.DMA],
    )
    return f(td, ts, xq, dst, src, wdp)


def _edge_body(go_ref, ea_ref, wea_ref, be1_ref, we2_ref,
               be2_ref, wc1_ref, bc1_ref, wc2p_ref, bc2_ref,
               mo_ref, wct_ref):
    g = go_ref[...]
    pre1 = (g + jnp.dot(ea_ref[...], wea_ref[...],
                        preferred_element_type=jnp.float32)
            + be1_ref[...])
    m1 = _silu(pre1)
    msg = _silu(jnp.dot(m1, we2_ref[...],
                        preferred_element_type=jnp.float32) + be2_ref[...])
    c1 = _silu(jnp.dot(msg, wc1_ref[...],
                       preferred_element_type=jnp.float32) + bc1_ref[...])
    wct = lax.dot_general(wc2p_ref[...], c1,
                          (((0,), (1,)), ((), ())),
                          preferred_element_type=jnp.float32)
    mo_ref[...] = msg
    wct_ref[...] = wct + bc2_ref[...]


def _edge_mlp(go, edge_attr, wea, be1, we2, be2, wc1, bc1, wc2p, bc2):
    return pl.pallas_call(
        _edge_body,
        grid=(NB_E,),
        in_specs=[
            pl.BlockSpec((BE, H), lambda i: (i, 0)),
            pl.BlockSpec((BE, EA), lambda i: (i, 0)),
            pl.BlockSpec((EA, H), lambda i: (0, 0)),
            pl.BlockSpec((1, H), lambda i: (0, 0)),
            pl.BlockSpec((H, H), lambda i: (0, 0)),
            pl.BlockSpec((1, H), lambda i: (0, 0)),
            pl.BlockSpec((H, H // 2), lambda i: (0, 0)),
            pl.BlockSpec((1, H // 2), lambda i: (0, 0)),
            pl.BlockSpec((H // 2, 8), lambda i: (0, 0)),
            pl.BlockSpec((1, 1), lambda i: (0, 0)),
        ],
        out_specs=[
            pl.BlockSpec((BE, H), lambda i: (i, 0)),
            pl.BlockSpec((8, BE), lambda i: (0, i)),
        ],
        out_shape=[
            jax.ShapeDtypeStruct((EH, H), jnp.float32),
            jax.ShapeDtypeStruct((8, EH), jnp.float32),
        ],
    )(go, edge_attr, wea, be1, we2, be2, wc1, bc1, wc2p, bc2)


ZR = 25
NPW = N // NS


def _rsqrt_nr(x):
    i = plsc.bitcast(x, jnp.int32)
    y = plsc.bitcast(jnp.int32(0x5F3759DF) - (i >> 1), jnp.float32)
    hx = 0.5 * x
    for _ in range(3):
        y = y * (1.5 - hx * y * y)
    return y


def _scatter_body(crow0, mo_hbm, wct_hbm, rq_hbm, dst2_hbm, outm_hbm,
                  outw_hbm,
                  dia, mbuf0, rqb0, wcb0, w4b0, mbuf1, rqb1, wcb1, w4b1,
                  zbuf, zwbuf,
                  lsem0, lsem1, ssem0, ssem1,
                  accm, accw):
    c = lax.axis_index("c")
    s = lax.axis_index("s")
    wid = c * NS + s
    base = wid * EPW
    pltpu.sync_copy(dst2_hbm.at[pl.ds(crow0 + wid * NCH, NCH)], dia)

    def zrow(i, carry):
        for k in range(H // 16):
            zbuf[i, pl.ds(k * 16, 16)] = jnp.zeros((16,), jnp.float32)
        zwbuf[i, :] = jnp.zeros((16,), jnp.float32)
        return carry

    lax.fori_loop(0, ZR, zrow, 0, unroll=2)
    r0 = s * NPW
    for j in range(NPW // ZR):
        pltpu.sync_copy(zbuf, accm.at[pl.ds(r0 + j * ZR, ZR)])
        pltpu.sync_copy(zwbuf, accw.at[pl.ds(r0 + j * ZR, ZR)])
    plsc.subcore_barrier()

    bufs = [(mbuf0, rqb0, wcb0, w4b0, lsem0, ssem0),
            (mbuf1, rqb1, wcb1, w4b1, lsem1, ssem1)]

    def fire_load(i, b):
        mb, rqb, wcb, _, ls, _ = bufs[b]
        e0 = base + i * CH
        pltpu.async_copy(mo_hbm.at[pl.ds(e0, CH)], mb, ls)
        pltpu.async_copy(rq_hbm.at[pl.ds(e0, CH)], rqb, ls)
        pltpu.async_copy(wct_hbm.at[0, pl.ds(e0, CH)], wcb.at[pl.ds(0, CH)],
                         ls)

    def drain_load(b):
        mb, rqb, wcb, _, ls, _ = bufs[b]
        pltpu.make_async_copy(mo_hbm.at[pl.ds(0, CH)], mb, ls).wait()
        pltpu.make_async_copy(rq_hbm.at[pl.ds(0, CH)], rqb, ls).wait()
        pltpu.make_async_copy(wct_hbm.at[0, pl.ds(0, CH)],
                              wcb.at[pl.ds(0, CH)], ls).wait()

    ii = lax.iota(jnp.int32, 16)
    e3 = jnp.where(ii == 3, 1.0, 0.0).astype(jnp.float32)

    def compute(b):
        mb, rqb, wcb, w4b, _, _ = bufs[b]

        def edge(e, c2):
            v = rqb[e, :]
            ds = jnp.full((16,), jnp.maximum(jnp.sum(v * v), 1e-8),
                          jnp.float32)
            d = ds * _rsqrt_nr(ds)
            z = d + 1.0
            inv = _rsqrt_nr(z * z)
            wcv = wcb[pl.ds(e, 16)]
            w4b[e, :] = v * (inv * wcv[0]) + e3
            return c2

        lax.fori_loop(0, CH, edge, 0, unroll=2)

    def fire_scatter(i, b):
        mb, _, _, w4b, _, ss = bufs[b]
        pltpu.async_copy(mb, accm.at[dia.at[i]], ss, add=True)
        pltpu.async_copy(w4b, accw.at[dia.at[i]], ss, add=True)

    def drain_scatter(b):
        mb, _, _, w4b, _, ss = bufs[b]
        pltpu.make_async_copy(mb, accm.at[pl.ds(0, CH)], ss).wait()
        pltpu.make_async_copy(w4b, accw.at[pl.ds(0, CH)], ss).wait()

    fire_load(0, 0)

    @pl.loop(0, (NCH - 1) // 2)
    def body(p):
        i0 = 2 * p
        fire_load(i0 + 1, 1)
        drain_load(0)
        compute(0)
        fire_scatter(i0, 0)
        drain_scatter(0)
        fire_load(i0 + 2, 0)
        drain_load(1)
        compute(1)
        fire_scatter(i0 + 1, 1)
        drain_scatter(1)

    drain_load(0)
    compute(0)
    fire_scatter(NCH - 1, 0)
    drain_scatter(0)

    plsc.subcore_barrier()
    for j in range(NPW // ZR):
        rr = r0 + j * ZR
        pltpu.sync_copy(accm.at[pl.ds(rr, ZR)], outm_hbm.at[c, pl.ds(rr, ZR)])
        pltpu.sync_copy(accw.at[pl.ds(rr, ZR)], outw_hbm.at[c, pl.ds(rr, ZR)])


def _sc_scatter(mo, wct, rq, dst2, crow0):
    mesh = plsc.VectorSubcoreMesh(core_axis_name="c", subcore_axis_name="s")
    f = pl.kernel(
        functools.partial(_scatter_body, crow0),
        mesh=mesh,
        compiler_params=pltpu.CompilerParams(use_tc_tiling_on_sc=False,
                                             needs_layout_passes=False),
        out_type=[
            jax.ShapeDtypeStruct((NC, N, H), jnp.float32),
            jax.ShapeDtypeStruct((NC, N, 16), jnp.float32),
        ],
        scratch_types=[
            pltpu.VMEM((NCH, CH), jnp.int32),
        ] + 2 * [
            pltpu.VMEM((CH, H), jnp.float32),
            pltpu.VMEM((CH, 16), jnp.float32),
            pltpu.VMEM((CH + 16,), jnp.float32),
            pltpu.VMEM((CH, 16), jnp.float32),
        ] + [
            pltpu.VMEM((ZR, H), jnp.float32),
            pltpu.VMEM((ZR, 16), jnp.float32),
        ] + 4 * [pltpu.SemaphoreType.DMA] + [
            pltpu.VMEM_SHARED((N, H), jnp.float32),
            pltpu.VMEM_SHARED((N, 16), jnp.float32),
        ],
    )
    return f(mo, wct, rq, dst2)


def _node_body(h_ref, xp_ref, a0_ref, a1_ref, a2_ref, a3_ref,
               c0_ref, c1_ref, c2_ref, c3_ref, wn1h_ref,
               wn1a_ref, bn1_ref, wn2_ref, bn2_ref, lng_ref, lnb_ref, sl_ref,
               h_out_ref, x_out_ref):
    hb = h_ref[...]
    agg = (a0_ref[...] + a1_ref[...]) + (a2_ref[...] + a3_ref[...])
    cacc = (c0_ref[...] + c1_ref[...]) + (c2_ref[...] + c3_ref[...])
    t = _silu(jnp.dot(hb, wn1h_ref[...], preferred_element_type=jnp.float32)
              + jnp.dot(agg, wn1a_ref[...],
                        preferred_element_type=jnp.float32)
              + bn1_ref[...])
    ho = (jnp.dot(t, wn2_ref[...], preferred_element_type=jnp.float32)
          + bn2_ref[...] + hb)
    mu = jnp.mean(ho, axis=-1, keepdims=True)
    ctr = ho - mu
    var = jnp.mean(ctr * ctr, axis=-1, keepdims=True)
    h_out_ref[...] = ctr * lax.rsqrt(var + 1e-5) * lng_ref[...] + lnb_ref[...]
    deg = jnp.clip(cacc[:, 3:4], 1.0, None)
    scale = jax.nn.sigmoid(sl_ref[0, 0])
    x_out_ref[...] = xp_ref[...] + scale * (cacc / deg)


def _node_mlp(h, xp, aas, ccs, wn1h, wn1a, bn1, wn2, bn2, lng, lnb, sl):
    return pl.pallas_call(
        _node_body,
        grid=(NB_N,),
        in_specs=[
            pl.BlockSpec((BN, D), lambda i: (i, 0)),
            pl.BlockSpec((BN, 16), lambda i: (i, 0)),
        ] + 4 * [
            pl.BlockSpec((BN, H), lambda i: (i, 0)),
        ] + 4 * [
            pl.BlockSpec((BN, 16), lambda i: (i, 0)),
        ] + [
            pl.BlockSpec((D, H), lambda i: (0, 0)),
            pl.BlockSpec((H, H), lambda i: (0, 0)),
            pl.BlockSpec((1, H), lambda i: (0, 0)),
            pl.BlockSpec((H, D), lambda i: (0, 0)),
            pl.BlockSpec((1, D), lambda i: (0, 0)),
            pl.BlockSpec((1, D), lambda i: (0, 0)),
            pl.BlockSpec((1, D), lambda i: (0, 0)),
            pl.BlockSpec((1, 1), lambda i: (0, 0)),
        ],
        out_specs=[
            pl.BlockSpec((BN, D), lambda i: (i, 0)),
            pl.BlockSpec((BN, 16), lambda i: (i, 0)),
        ],
        out_shape=[
            jax.ShapeDtypeStruct((N, D), jnp.float32),
            jax.ShapeDtypeStruct((N, 16), jnp.float32),
        ],
    )(h, xp, *aas, *ccs, wn1h, wn1a, bn1, wn2, bn2, lng, lnb, sl)


def kernel(h, x, edge_index, edge_attr, We1, be1, We2, be2, Wn1, bn1,
           Wn2, bn2, Wc1, bc1, Wc2, bc2, ln_g, ln_b, scale_logit):
    dst = edge_index[1]
    src = edge_index[0]
    xp = jnp.pad(x, ((0, 0), (0, 13)))

    we1a = We1[:D]
    we1b = We1[D:2 * D]
    wd = We1[2 * D:2 * D + 1]
    wea = We1[2 * D + 1:]
    pd, ps = _project(h, we1a, we1b)

    dst2 = dst.reshape(E // CH, CH)
    src2 = src.reshape(E // CH, CH)

    wdp = wd[0, PERM]
    wc2p = jnp.pad(Wc2, ((0, 0), (0, 7)))

    aas, ccs = [], []
    for j in range(SPL):
        go, rq = _sc_gather(pd, ps, xp, dst2, src2, wdp, j * CPS)
        mo, wct = _edge_mlp(go, edge_attr[j * EH:(j + 1) * EH],
                            wea[:, PERM],
                            be1[PERM].reshape(1, H), We2[PERM, :],
                            be2.reshape(1, H), Wc1, bc1.reshape(1, H // 2),
                            wc2p, bc2.reshape(1, 1))
        accm, accw = _sc_scatter(mo, wct, rq, dst2, j * CPS)
        aas += [accm[0], accm[1]]
        ccs += [accw[0], accw[1]]

    h_out, x_out16 = _node_mlp(
        h, xp, aas, ccs, Wn1[:D], Wn1[D:],
        bn1.reshape(1, H), Wn2, bn2.reshape(1, D), ln_g.reshape(1, D),
        ln_b.reshape(1, D), scale_logit.reshape(1, 1))
    return (h_out, x_out16[:, :3])


# speedup vs baseline: 1.6481x; 1.6481x over previous
import functools

import numpy as np

import jax
import jax.numpy as jnp
from jax import lax
from jax.experimental import pallas as pl
from jax.experimental.pallas import tpu as pltpu
from jax.experimental.pallas import tpu_sc as plsc

N = 10000
E = 320000
D = 128
H = 128
EA = 4
W = 144
NB_N = 10
BN = N // NB_N
SPL = 2
EH = E // SPL
BE = 3200
NB_E = EH // BE

PERM = np.concatenate(
    [np.concatenate([np.arange(16) * 2 + 32 * k,
                     np.arange(16) * 2 + 1 + 32 * k]) for k in range(4)])

NC = 2
NS = 16
NW = NC * NS
EPW = EH // NW
CH = 40
NCH = EPW // CH
CPS = EH // CH


def _silu(v):
    return v * jax.nn.sigmoid(v)


def _proj_body(h_ref, wa_ref, wb_ref, pd_ref, ps_ref):
    hb = h_ref[...]
    pd_ref[...] = jnp.dot(
        hb, wa_ref[...], preferred_element_type=jnp.float32
    ).astype(jnp.bfloat16)
    ps_ref[...] = jnp.dot(
        hb, wb_ref[...], preferred_element_type=jnp.float32
    ).astype(jnp.bfloat16)


def _project(h, we1a, we1b):
    return pl.pallas_call(
        _proj_body,
        grid=(NB_N,),
        in_specs=[
            pl.BlockSpec((BN, D), lambda i: (i, 0)),
            pl.BlockSpec((D, H), lambda i: (0, 0)),
            pl.BlockSpec((D, H), lambda i: (0, 0)),
        ],
        out_specs=[
            pl.BlockSpec((BN, H), lambda i: (i, 0)),
            pl.BlockSpec((BN, H), lambda i: (i, 0)),
        ],
        out_shape=[
            jax.ShapeDtypeStruct((N, H), jnp.bfloat16),
            jax.ShapeDtypeStruct((N, H), jnp.bfloat16),
        ],
    )(h, we1a, we1b)


def _gather_body(crow0, td_hbm, ts_hbm, xq_hbm, dst2_hbm, src2_hbm,
                 go_hbm, rq_hbm, ds_hbm,
                 dia, sia,
                 dbuf0, sbuf0, xd0, xs0, obuf0, rbuf0, qbuf0,
                 dbuf1, sbuf1, xd1, xs1, obuf1, rbuf1, qbuf1,
                 gsem0, gsem1, wsem0, wsem1):
    wid = lax.axis_index("s") * NC + lax.axis_index("c")
    base = wid * EPW
    pltpu.sync_copy(dst2_hbm.at[pl.ds(crow0 + wid * NCH, NCH)], dia)
    pltpu.sync_copy(src2_hbm.at[pl.ds(crow0 + wid * NCH, NCH)], sia)

    bufs = [(dbuf0, sbuf0, xd0, xs0, obuf0, rbuf0, qbuf0, gsem0, wsem0),
            (dbuf1, sbuf1, xd1, xs1, obuf1, rbuf1, qbuf1, gsem1, wsem1)]

    def fire(i, b):
        db, sb, xd, xs, _, _, _, gs, _ = bufs[b]
        di = dia.at[i]
        si = sia.at[i]
        pltpu.async_copy(td_hbm.at[di], db, gs)
        pltpu.async_copy(ts_hbm.at[si], sb, gs)
        pltpu.async_copy(xq_hbm.at[di], xd, gs)
        pltpu.async_copy(xq_hbm.at[si], xs, gs)

    def drain_g(b):
        db, sb, xd, xs, _, _, _, gs, _ = bufs[b]
        pltpu.make_async_copy(td_hbm.at[pl.ds(0, CH)], db, gs).wait()
        pltpu.make_async_copy(ts_hbm.at[pl.ds(0, CH)], sb, gs).wait()
        pltpu.make_async_copy(xq_hbm.at[pl.ds(0, CH)], xd, gs).wait()
        pltpu.make_async_copy(xq_hbm.at[pl.ds(0, CH)], xs, gs).wait()

    def drain_w(b):
        _, _, _, _, ob, rb, qb, _, ws = bufs[b]
        pltpu.make_async_copy(ob, go_hbm.at[pl.ds(0, CH)], ws).wait()
        pltpu.make_async_copy(rb, rq_hbm.at[pl.ds(0, CH)], ws).wait()
        pltpu.make_async_copy(qb, ds_hbm.at[pl.ds(0, CH)], ws).wait()

    def compute(b):
        db, sb, xd, xs, ob, rb, qb, _, _ = bufs[b]

        himask = jnp.int32(-65536)

        def row(r, c2):
            rel = xd[r, :] - xs[r, :]
            rb[r, :] = rel
            qbuf = rel * rel
            qb[r, :] = jnp.full(
                (16,), jnp.maximum(qbuf[0] + qbuf[1] + qbuf[2], 1e-8),
                jnp.float32)
            for k in range(4):
                vd = plsc.bitcast(db[r, pl.ds(k * 32, 32)], jnp.int32)
                vs = plsc.bitcast(sb[r, pl.ds(k * 32, 32)], jnp.int32)
                ob[r, pl.ds(k * 32, 16)] = (
                    plsc.bitcast(vd << 16, jnp.float32)
                    + plsc.bitcast(vs << 16, jnp.float32))
                ob[r, pl.ds(k * 32 + 16, 16)] = (
                    plsc.bitcast(vd & himask, jnp.float32)
                    + plsc.bitcast(vs & himask, jnp.float32))
            return c2

        lax.fori_loop(0, CH, row, 0, unroll=2)

    def write(i, b):
        _, _, _, _, ob, rb, qb, _, ws = bufs[b]
        e0 = base + i * CH
        pltpu.async_copy(ob, go_hbm.at[pl.ds(e0, CH)], ws)
        pltpu.async_copy(rb, rq_hbm.at[pl.ds(e0, CH)], ws)
        pltpu.async_copy(qb, ds_hbm.at[pl.ds(e0, CH)], ws)

    fire(0, 0)

    @pl.loop(0, (NCH - 1) // 2)
    def body(p):
        i0 = 2 * p
        fire(i0 + 1, 1)
        drain_g(0)

        @pl.when(i0 >= 2)
        def _():
            drain_w(0)

        compute(0)
        write(i0, 0)
        fire(i0 + 2, 0)
        drain_g(1)

        @pl.when(i0 >= 1)
        def _():
            drain_w(1)

        compute(1)
        write(i0 + 1, 1)

    drain_g(0)
    drain_w(0)
    compute(0)
    write(NCH - 1, 0)
    drain_w(0)
    drain_w(1)


def _sc_gather(td, ts, xq, dst, src, crow0):
    mesh = plsc.VectorSubcoreMesh(core_axis_name="c", subcore_axis_name="s")
    f = pl.kernel(
        functools.partial(_gather_body, crow0),
        mesh=mesh,
        compiler_params=pltpu.CompilerParams(use_tc_tiling_on_sc=False,
                                             needs_layout_passes=False),
        out_type=[
            jax.ShapeDtypeStruct((EH, H), jnp.float32),
            jax.ShapeDtypeStruct((EH, 16), jnp.float32),
            jax.ShapeDtypeStruct((EH, 16), jnp.float32),
        ],
        scratch_types=[
            pltpu.VMEM((NCH, CH), jnp.int32),
            pltpu.VMEM((NCH, CH), jnp.int32),
        ] + 2 * [
            pltpu.VMEM((CH, H), jnp.bfloat16),
            pltpu.VMEM((CH, H), jnp.bfloat16),
            pltpu.VMEM((CH, 16), jnp.float32),
            pltpu.VMEM((CH, 16), jnp.float32),
            pltpu.VMEM((CH, H), jnp.float32),
            pltpu.VMEM((CH, 16), jnp.float32),
            pltpu.VMEM((CH, 16), jnp.float32),
        ] + 4 * [pltpu.SemaphoreType.DMA],
    )
    return f(td, ts, xq, dst, src)


def _edge_body(go_ref, ds_ref, ea_ref, wea_ref, wd_ref, be1_ref, we2_ref,
               be2_ref, wc1_ref, bc1_ref, wc2_ref, bc2_ref,
               mo_ref, wct_ref):
    g = go_ref[...]
    dist_sq = ds_ref[...][:, 0:1]
    pre1 = (g + dist_sq * wd_ref[...]
            + jnp.dot(ea_ref[...], wea_ref[...],
                      preferred_element_type=jnp.float32)
            + be1_ref[...])
    m1 = _silu(pre1)
    msg = _silu(jnp.dot(m1, we2_ref[...],
                        preferred_element_type=jnp.float32) + be2_ref[...])
    c1 = _silu(jnp.dot(msg, wc1_ref[...],
                       preferred_element_type=jnp.float32) + bc1_ref[...])
    wc = jnp.dot(c1, wc2_ref[...],
                 preferred_element_type=jnp.float32) + bc2_ref[...]
    swc = wc / (jnp.sqrt(dist_sq) + 1.0)
    ones8 = jnp.ones((1, 8), jnp.float32)
    mo_ref[...] = msg
    wct_ref[...] = lax.dot_general(ones8, swc,
                                   (((0,), (1,)), ((), ())),
                                   preferred_element_type=jnp.float32)


def _edge_mlp(go, dsq, edge_attr, wea, wd, be1, we2, be2, wc1, bc1, wc2, bc2):
    return pl.pallas_call(
        _edge_body,
        grid=(NB_E,),
        in_specs=[
            pl.BlockSpec((BE, H), lambda i: (i, 0)),
            pl.BlockSpec((BE, 16), lambda i: (i, 0)),
            pl.BlockSpec((BE, EA), lambda i: (i, 0)),
            pl.BlockSpec((EA, H), lambda i: (0, 0)),
            pl.BlockSpec((1, H), lambda i: (0, 0)),
            pl.BlockSpec((1, H), lambda i: (0, 0)),
            pl.BlockSpec((H, H), lambda i: (0, 0)),
            pl.BlockSpec((1, H), lambda i: (0, 0)),
            pl.BlockSpec((H, H // 2), lambda i: (0, 0)),
            pl.BlockSpec((1, H // 2), lambda i: (0, 0)),
            pl.BlockSpec((H // 2, 1), lambda i: (0, 0)),
            pl.BlockSpec((1, 1), lambda i: (0, 0)),
        ],
        out_specs=[
            pl.BlockSpec((BE, H), lambda i: (i, 0)),
            pl.BlockSpec((8, BE), lambda i: (0, i)),
        ],
        out_shape=[
            jax.ShapeDtypeStruct((EH, H), jnp.float32),
            jax.ShapeDtypeStruct((8, EH), jnp.float32),
        ],
    )(go, dsq, edge_attr, wea, wd, be1, we2, be2, wc1, bc1, wc2, bc2)


ZR = 25
NPW = N // NS


def _scatter_body(crow0, mo_hbm, wct_hbm, rq_hbm, dst2_hbm, outm_hbm,
                  outw_hbm,
                  dia, mbuf0, rqb0, wcb0, w4b0, mbuf1, rqb1, wcb1, w4b1,
                  zbuf, zwbuf,
                  lsem0, lsem1, ssem0, ssem1,
                  accm, accw):
    c = lax.axis_index("c")
    s = lax.axis_index("s")
    wid = c * NS + s
    base = wid * EPW
    pltpu.sync_copy(dst2_hbm.at[pl.ds(crow0 + wid * NCH, NCH)], dia)

    def zrow(i, carry):
        for k in range(H // 16):
            zbuf[i, pl.ds(k * 16, 16)] = jnp.zeros((16,), jnp.float32)
        zwbuf[i, :] = jnp.zeros((16,), jnp.float32)
        return carry

    lax.fori_loop(0, ZR, zrow, 0, unroll=2)
    r0 = s * NPW
    for j in range(NPW // ZR):
        pltpu.sync_copy(zbuf, accm.at[pl.ds(r0 + j * ZR, ZR)])
        pltpu.sync_copy(zwbuf, accw.at[pl.ds(r0 + j * ZR, ZR)])
    plsc.subcore_barrier()

    bufs = [(mbuf0, rqb0, wcb0, w4b0, lsem0, ssem0),
            (mbuf1, rqb1, wcb1, w4b1, lsem1, ssem1)]

    def fire_load(i, b):
        mb, rqb, wcb, _, ls, _ = bufs[b]
        e0 = base + i * CH
        pltpu.async_copy(mo_hbm.at[pl.ds(e0, CH)], mb, ls)
        pltpu.async_copy(rq_hbm.at[pl.ds(e0, CH)], rqb, ls)
        pltpu.async_copy(wct_hbm.at[0, pl.ds(e0, CH)], wcb.at[pl.ds(0, CH)],
                         ls)

    def drain_load(b):
        mb, rqb, wcb, _, ls, _ = bufs[b]
        pltpu.make_async_copy(mo_hbm.at[pl.ds(0, CH)], mb, ls).wait()
        pltpu.make_async_copy(rq_hbm.at[pl.ds(0, CH)], rqb, ls).wait()
        pltpu.make_async_copy(wct_hbm.at[0, pl.ds(0, CH)],
                              wcb.at[pl.ds(0, CH)], ls).wait()

    ii = lax.iota(jnp.int32, 16)
    e3 = jnp.where(ii == 3, 1.0, 0.0).astype(jnp.float32)

    def compute(b):
        mb, rqb, wcb, w4b, _, _ = bufs[b]

        def edge(e, c2):
            wcv = wcb[pl.ds(e, 16)]
            w4b[e, :] = rqb[e, :] * wcv[0] + e3
            return c2

        lax.fori_loop(0, CH, edge, 0, unroll=2)

    def fire_scatter(i, b):
        mb, _, _, w4b, _, ss = bufs[b]
        pltpu.async_copy(mb, accm.at[dia.at[i]], ss, add=True)
        pltpu.async_copy(w4b, accw.at[dia.at[i]], ss, add=True)

    def drain_scatter(b):
        mb, _, _, w4b, _, ss = bufs[b]
        pltpu.make_async_copy(mb, accm.at[pl.ds(0, CH)], ss).wait()
        pltpu.make_async_copy(w4b, accw.at[pl.ds(0, CH)], ss).wait()

    fire_load(0, 0)

    @pl.loop(0, (NCH - 1) // 2)
    def body(p):
        i0 = 2 * p
        fire_load(i0 + 1, 1)
        drain_load(0)
        compute(0)
        fire_scatter(i0, 0)
        drain_scatter(0)
        fire_load(i0 + 2, 0)
        drain_load(1)
        compute(1)
        fire_scatter(i0 + 1, 1)
        drain_scatter(1)

    drain_load(0)
    compute(0)
    fire_scatter(NCH - 1, 0)
    drain_scatter(0)

    plsc.subcore_barrier()
    for j in range(NPW // ZR):
        rr = r0 + j * ZR
        pltpu.sync_copy(accm.at[pl.ds(rr, ZR)], outm_hbm.at[c, pl.ds(rr, ZR)])
        pltpu.sync_copy(accw.at[pl.ds(rr, ZR)], outw_hbm.at[c, pl.ds(rr, ZR)])


def _sc_scatter(mo, wct, rq, dst2, crow0):
    mesh = plsc.VectorSubcoreMesh(core_axis_name="c", subcore_axis_name="s")
    f = pl.kernel(
        functools.partial(_scatter_body, crow0),
        mesh=mesh,
        compiler_params=pltpu.CompilerParams(use_tc_tiling_on_sc=False,
                                             needs_layout_passes=False),
        out_type=[
            jax.ShapeDtypeStruct((NC, N, H), jnp.float32),
            jax.ShapeDtypeStruct((NC, N, 16), jnp.float32),
        ],
        scratch_types=[
            pltpu.VMEM((NCH, CH), jnp.int32),
        ] + 2 * [
            pltpu.VMEM((CH, H), jnp.float32),
            pltpu.VMEM((CH, 16), jnp.float32),
            pltpu.VMEM((CH + 16,), jnp.float32),
            pltpu.VMEM((CH, 16), jnp.float32),
        ] + [
            pltpu.VMEM((ZR, H), jnp.float32),
            pltpu.VMEM((ZR, 16), jnp.float32),
        ] + 4 * [pltpu.SemaphoreType.DMA] + [
            pltpu.VMEM_SHARED((N, H), jnp.float32),
            pltpu.VMEM_SHARED((N, 16), jnp.float32),
        ],
    )
    return f(mo, wct, rq, dst2)


def _node_body(h_ref, xp_ref, a0_ref, a1_ref, a2_ref, a3_ref,
               c0_ref, c1_ref, c2_ref, c3_ref, wn1h_ref,
               wn1a_ref, bn1_ref, wn2_ref, bn2_ref, lng_ref, lnb_ref, sl_ref,
               h_out_ref, x_out_ref):
    hb = h_ref[...]
    agg = (a0_ref[...] + a1_ref[...]) + (a2_ref[...] + a3_ref[...])
    cacc = (c0_ref[...] + c1_ref[...]) + (c2_ref[...] + c3_ref[...])
    t = _silu(jnp.dot(hb, wn1h_ref[...], preferred_element_type=jnp.float32)
              + jnp.dot(agg, wn1a_ref[...],
                        preferred_element_type=jnp.float32)
              + bn1_ref[...])
    ho = (jnp.dot(t, wn2_ref[...], preferred_element_type=jnp.float32)
          + bn2_ref[...] + hb)
    mu = jnp.mean(ho, axis=-1, keepdims=True)
    ctr = ho - mu
    var = jnp.mean(ctr * ctr, axis=-1, keepdims=True)
    h_out_ref[...] = ctr * lax.rsqrt(var + 1e-5) * lng_ref[...] + lnb_ref[...]
    deg = jnp.clip(cacc[:, 3:4], 1.0, None)
    scale = jax.nn.sigmoid(sl_ref[0, 0])
    x_out_ref[...] = xp_ref[...] + scale * (cacc / deg)


def _node_mlp(h, xp, aas, ccs, wn1h, wn1a, bn1, wn2, bn2, lng, lnb, sl):
    return pl.pallas_call(
        _node_body,
        grid=(NB_N,),
        in_specs=[
            pl.BlockSpec((BN, D), lambda i: (i, 0)),
            pl.BlockSpec((BN, 16), lambda i: (i, 0)),
        ] + 4 * [
            pl.BlockSpec((BN, H), lambda i: (i, 0)),
        ] + 4 * [
            pl.BlockSpec((BN, 16), lambda i: (i, 0)),
        ] + [
            pl.BlockSpec((D, H), lambda i: (0, 0)),
            pl.BlockSpec((H, H), lambda i: (0, 0)),
            pl.BlockSpec((1, H), lambda i: (0, 0)),
            pl.BlockSpec((H, D), lambda i: (0, 0)),
            pl.BlockSpec((1, D), lambda i: (0, 0)),
            pl.BlockSpec((1, D), lambda i: (0, 0)),
            pl.BlockSpec((1, D), lambda i: (0, 0)),
            pl.BlockSpec((1, 1), lambda i: (0, 0)),
        ],
        out_specs=[
            pl.BlockSpec((BN, D), lambda i: (i, 0)),
            pl.BlockSpec((BN, 16), lambda i: (i, 0)),
        ],
        out_shape=[
            jax.ShapeDtypeStruct((N, D), jnp.float32),
            jax.ShapeDtypeStruct((N, 16), jnp.float32),
        ],
    )(h, xp, *aas, *ccs, wn1h, wn1a, bn1, wn2, bn2, lng, lnb, sl)


def kernel(h, x, edge_index, edge_attr, We1, be1, We2, be2, Wn1, bn1,
           Wn2, bn2, Wc1, bc1, Wc2, bc2, ln_g, ln_b, scale_logit):
    dst = edge_index[1]
    src = edge_index[0]
    xp = jnp.pad(x, ((0, 0), (0, 13)))

    we1a = We1[:D]
    we1b = We1[D:2 * D]
    wd = We1[2 * D:2 * D + 1]
    wea = We1[2 * D + 1:]
    pd, ps = _project(h, we1a, we1b)

    dst2 = dst.reshape(E // CH, CH)
    src2 = src.reshape(E // CH, CH)

    aas, ccs = [], []
    for j in range(SPL):
        go, rq, dsq = _sc_gather(pd, ps, xp, dst2, src2, j * CPS)
        mo, wct = _edge_mlp(go, dsq, edge_attr[j * EH:(j + 1) * EH],
                            wea[:, PERM], wd[:, PERM],
                            be1[PERM].reshape(1, H), We2[PERM, :],
                            be2.reshape(1, H), Wc1, bc1.reshape(1, H // 2),
                            Wc2, bc2.reshape(1, 1))
        accm, accw = _sc_scatter(mo, wct, rq, dst2, j * CPS)
        aas += [accm[0], accm[1]]
        ccs += [accw[0], accw[1]]

    h_out, x_out16 = _node_mlp(
        h, xp, aas, ccs, Wn1[:D], Wn1[D:],
        bn1.reshape(1, H), Wn2, bn2.reshape(1, D), ln_g.reshape(1, D),
        ln_b.reshape(1, D), scale_logit.reshape(1, 1))
    return (h_out, x_out16[:, :3])
